# baseline (speedup 1.0000x reference)
import functools

import jax
import jax.numpy as jnp
from jax.experimental import pallas as pl
from jax.experimental.pallas import tpu as pltpu

S, D = 2048, 2048
H, KVH, DH = 16, 4, 128
E, TOPK, F = 8, 2, 768
EPS = 1e-6
THETA = 1000000.0

HIGH = jax.lax.Precision.HIGHEST


def _rms(x, w, eps=EPS):
    v = jnp.mean(jnp.square(x), axis=-1, keepdims=True)
    return x * jax.lax.rsqrt(v + eps) * w


def _qkv_kernel(x_ref, lnw_ref, w_ref, o_ref):
    x = x_ref[...]
    xn = _rms(x, lnw_ref[...])
    o_ref[...] = jnp.dot(xn, w_ref[...], precision=None,
                         preferred_element_type=jnp.float32)


def _qkv(x, lnw, wqkv, br=256, bc=768):
    nr, nc = S // br, (H * DH + 2 * KVH * DH) // bc
    return pl.pallas_call(
        _qkv_kernel,
        grid=(nc, nr),
        in_specs=[
            pl.BlockSpec((br, D), lambda c, r: (r, 0)),
            pl.BlockSpec((1, D), lambda c, r: (0, 0)),
            pl.BlockSpec((D, bc), lambda c, r: (0, c)),
        ],
        out_specs=pl.BlockSpec((br, bc), lambda c, r: (r, c)),
        out_shape=jax.ShapeDtypeStruct((S, H * DH + 2 * KVH * DH), jnp.float32),
    )(x, lnw, wqkv)


def _rope_cos_sin(base_row, nrows):
    sh = (nrows, DH // 2)
    pos = (base_row +
           jax.lax.broadcasted_iota(jnp.int32, sh, 0)).astype(jnp.float32)
    inv = 1.0 / (THETA ** (
        jax.lax.broadcasted_iota(jnp.int32, sh, 1).astype(jnp.float32)
        * (2.0 / DH)))
    ang = pos * inv
    return jnp.cos(ang), jnp.sin(ang)


def _norm_rope(x, w, base_row, nrows):
    xn = _rms(x, w)
    c, s = _rope_cos_sin(base_row, nrows)
    x1 = xn[:, :DH // 2]
    x2 = xn[:, DH // 2:]
    return jnp.concatenate([x1 * c - x2 * s, x2 * c + x1 * s], axis=-1)


def _flash_kernel(q_ref, k_ref, v_ref, qw_ref, kw_ref, o_ref,
                  acc_ref, m_ref, l_ref, *, bq, bk, nk):
    qb = pl.program_id(1)
    kb = pl.program_id(2)

    @pl.when(kb == 0)
    def _():
        acc_ref[...] = jnp.zeros_like(acc_ref)
        m_ref[...] = jnp.full_like(m_ref, -1e30)
        l_ref[...] = jnp.zeros_like(l_ref)

    @pl.when(kb <= qb)
    def _():
        kb_eff = jnp.minimum(kb, qb)
        q = _norm_rope(q_ref[...], qw_ref[...], qb * bq, bq)
        k = _norm_rope(k_ref[...], kw_ref[...], kb_eff * bk, bk)
        s = jax.lax.dot_general(q, k, (((1,), (1,)), ((), ())),
                                precision=None,
                                preferred_element_type=jnp.float32)
        s = s * (1.0 / (DH ** 0.5))

        @pl.when(kb == qb)
        def _():
            qi = jax.lax.broadcasted_iota(jnp.int32, (bq, bk), 0)
            ki = jax.lax.broadcasted_iota(jnp.int32, (bq, bk), 1)
            s2 = jnp.where(ki <= qi, s, -1e30)
            _flash_update(s2, v_ref, acc_ref, m_ref, l_ref)

        @pl.when(kb < qb)
        def _():
            _flash_update(s, v_ref, acc_ref, m_ref, l_ref)

    @pl.when(kb == nk - 1)
    def _():
        o_ref[...] = acc_ref[...] / l_ref[...][:, :1]


def _flash_update(s, v_ref, acc_ref, m_ref, l_ref):
    m_prev = m_ref[...][:, :1]
    m_cur = jnp.max(s, axis=-1, keepdims=True)
    m_new = jnp.maximum(m_prev, m_cur)
    alpha = jnp.exp(m_prev - m_new)
    p = jnp.exp(s - m_new)
    l_new = l_ref[...][:, :1] * alpha + jnp.sum(p, axis=-1, keepdims=True)
    l_ref[...] = jnp.broadcast_to(l_new, l_ref.shape)
    m_ref[...] = jnp.broadcast_to(m_new, m_ref.shape)
    pv = jnp.dot(p, v_ref[...], precision=None,
                 preferred_element_type=jnp.float32)
    acc_ref[...] = acc_ref[...] * alpha + pv


def _flash(qkv, qnw, knw, bq=256, bk=256):
    nq, nk = S // bq, S // bk
    kv0 = H
    v0 = H + KVH
    kern = functools.partial(_flash_kernel, bq=bq, bk=bk, nk=nk)
    return pl.pallas_call(
        kern,
        grid=(H, nq, nk),
        in_specs=[
            pl.BlockSpec((bq, DH), lambda h, qb, kb: (qb, h)),
            pl.BlockSpec((bk, DH),
                         lambda h, qb, kb: (jnp.minimum(kb, qb), kv0 + h // (H // KVH))),
            pl.BlockSpec((bk, DH),
                         lambda h, qb, kb: (jnp.minimum(kb, qb), v0 + h // (H // KVH))),
            pl.BlockSpec((1, DH), lambda h, qb, kb: (0, 0)),
            pl.BlockSpec((1, DH), lambda h, qb, kb: (0, 0)),
        ],
        out_specs=pl.BlockSpec((bq, DH), lambda h, qb, kb: (qb, h)),
        out_shape=jax.ShapeDtypeStruct((S, H * DH), jnp.float32),
        scratch_shapes=[
            pltpu.VMEM((bq, DH), jnp.float32),
            pltpu.VMEM((bq, 128), jnp.float32),
            pltpu.VMEM((bq, 128), jnp.float32),
        ],
    )(qkv, qkv, qkv, qnw, knw)


def _oproj_kernel(a_ref, x_ref, ow_ref, pw_ref, rw_ref,
                  res_ref, hn_ref, lg_ref):
    ao = jnp.dot(a_ref[...], ow_ref[...], precision=None,
                 preferred_element_type=jnp.float32)
    h = x_ref[...] + ao
    res_ref[...] = h
    hn = _rms(h, pw_ref[...])
    hn_ref[...] = hn
    lg_ref[...] = jnp.dot(hn, rw_ref[...], precision=None,
                          preferred_element_type=jnp.float32)


def _oproj(attn, x, ow, pw, rw, br=256):
    nr = S // br
    return pl.pallas_call(
        _oproj_kernel,
        grid=(nr,),
        in_specs=[
            pl.BlockSpec((br, H * DH), lambda r: (r, 0)),
            pl.BlockSpec((br, D), lambda r: (r, 0)),
            pl.BlockSpec((H * DH, D), lambda r: (0, 0)),
            pl.BlockSpec((1, D), lambda r: (0, 0)),
            pl.BlockSpec((D, E), lambda r: (0, 0)),
        ],
        out_specs=[
            pl.BlockSpec((br, D), lambda r: (r, 0)),
            pl.BlockSpec((br, D), lambda r: (r, 0)),
            pl.BlockSpec((br, E), lambda r: (r, 0)),
        ],
        out_shape=[
            jax.ShapeDtypeStruct((S, D), jnp.float32),
            jax.ShapeDtypeStruct((S, D), jnp.float32),
            jax.ShapeDtypeStruct((S, E), jnp.float32),
        ],
    )(attn, x, ow, pw, rw)


def _moe_kernel(hn_ref, lg_ref, res_ref, g_ref, u_ref, d_ref,
                o_ref, acc_ref, *, bt):
    e = pl.program_id(1)

    x = hn_ref[...]
    g = jnp.dot(x, g_ref[0], preferred_element_type=jnp.float32)
    u = jnp.dot(x, u_ref[0], preferred_element_type=jnp.float32)
    hf = (g * jax.nn.sigmoid(g)) * u

    lg = lg_ref[...]
    mx = jnp.max(lg, axis=-1, keepdims=True)
    p = jnp.exp(lg - mx)
    p = p / jnp.sum(p, axis=-1, keepdims=True)
    v1 = jnp.max(p, axis=-1, keepdims=True)
    v2 = jnp.max(jnp.where(p >= v1, -1.0, p), axis=-1, keepdims=True)
    wsel = jnp.where(p >= v2, p, 0.0) / (v1 + v2)
    eid = jax.lax.broadcasted_iota(jnp.int32, (1, E), 1)
    w_e = jnp.sum(wsel * (eid == e).astype(jnp.float32), axis=-1,
                  keepdims=True)

    contrib = jnp.dot((hf * w_e).astype(jnp.bfloat16), d_ref[0],
                      preferred_element_type=jnp.float32)

    @pl.when(e == 0)
    def _():
        acc_ref[...] = contrib

    @pl.when(e > 0)
    def _():
        acc_ref[...] = acc_ref[...] + contrib

    @pl.when(e == E - 1)
    def _():
        o_ref[...] = acc_ref[...] + res_ref[...]


def _moe(hn_bf16, logits, res, gk, uk, dk, bt=512):
    nt = S // bt
    kern = functools.partial(_moe_kernel, bt=bt)
    return pl.pallas_call(
        kern,
        grid=(nt, E),
        in_specs=[
            pl.BlockSpec((bt, D), lambda t, e: (t, 0)),
            pl.BlockSpec((bt, E), lambda t, e: (t, 0)),
            pl.BlockSpec((bt, D), lambda t, e: (t, 0)),
            pl.BlockSpec((1, D, F), lambda t, e: (e, 0, 0)),
            pl.BlockSpec((1, D, F), lambda t, e: (e, 0, 0)),
            pl.BlockSpec((1, F, D), lambda t, e: (e, 0, 0)),
        ],
        out_specs=pl.BlockSpec((bt, D), lambda t, e: (t, 0)),
        out_shape=jax.ShapeDtypeStruct((S, D), jnp.float32),
        scratch_shapes=[pltpu.VMEM((bt, D), jnp.float32)],
    )(hn_bf16, logits, res, gk, uk, dk)


def kernel(hidden_states, input_ln_w, q_w, k_w, v_w, o_w, q_norm_w,
           k_norm_w, post_ln_w, router_w, gate_k, up_k, down_k):
    x = hidden_states.reshape(S, D)
    wqkv = jnp.concatenate([q_w, k_w, v_w], axis=1)

    qkv = _qkv(x, input_ln_w.reshape(1, D), wqkv)
    attn = _flash(qkv, q_norm_w.reshape(1, DH), k_norm_w.reshape(1, DH))
    res2, hn, logits = _oproj(attn, x, o_w, post_ln_w.reshape(1, D), router_w)
    out = _moe(hn.astype(jnp.bfloat16), logits, res2,
               gate_k.astype(jnp.bfloat16), up_k.astype(jnp.bfloat16),
               down_k.astype(jnp.bfloat16))
    return out.reshape(1, S, D)


# speedup vs baseline: 2.8138x; 2.8138x over previous
import functools

import jax
import jax.numpy as jnp
from jax.experimental import pallas as pl
from jax.experimental.pallas import tpu as pltpu

S, D = 2048, 2048
H, KVH, DH = 16, 4, 128
E, TOPK, F = 8, 2, 768
EPS = 1e-6
THETA = 1000000.0

HIGH = jax.lax.Precision.HIGHEST


def _rms(x, w, eps=EPS):
    v = jnp.mean(jnp.square(x), axis=-1, keepdims=True)
    return x * jax.lax.rsqrt(v + eps) * w


def _qkv_kernel(x_ref, lnw_ref, w_ref, o_ref):
    x = x_ref[...]
    xn = _rms(x, lnw_ref[...])
    o_ref[...] = jnp.dot(xn, w_ref[...], precision=None,
                         preferred_element_type=jnp.float32)


def _qkv(x, lnw, wqkv, br=256, bc=768):
    nr, nc = S // br, (H * DH + 2 * KVH * DH) // bc
    return pl.pallas_call(
        _qkv_kernel,
        grid=(nc, nr),
        in_specs=[
            pl.BlockSpec((br, D), lambda c, r: (r, 0)),
            pl.BlockSpec((1, D), lambda c, r: (0, 0)),
            pl.BlockSpec((D, bc), lambda c, r: (0, c)),
        ],
        out_specs=pl.BlockSpec((br, bc), lambda c, r: (r, c)),
        out_shape=jax.ShapeDtypeStruct((S, H * DH + 2 * KVH * DH), jnp.float32),
    )(x, lnw, wqkv)


def _rope_cos_sin(base_row, nrows):
    sh = (nrows, DH // 2)
    pos = (base_row +
           jax.lax.broadcasted_iota(jnp.int32, sh, 0)).astype(jnp.float32)
    inv = 1.0 / (THETA ** (
        jax.lax.broadcasted_iota(jnp.int32, sh, 1).astype(jnp.float32)
        * (2.0 / DH)))
    ang = pos * inv
    return jnp.cos(ang), jnp.sin(ang)


def _norm_rope(x, w, base_row, nrows):
    xn = _rms(x, w)
    c, s = _rope_cos_sin(base_row, nrows)
    x1 = xn[:, :DH // 2]
    x2 = xn[:, DH // 2:]
    return jnp.concatenate([x1 * c - x2 * s, x2 * c + x1 * s], axis=-1)


def _flash_kernel(q_ref, k_ref, v_ref, qw_ref, kw_ref, o_ref, kn_ref,
                  *, bq, bk):
    qb = pl.program_id(1)
    gh = H // KVH

    @pl.when(qb == 0)
    def _():
        kn_ref[...] = _norm_rope(k_ref[...], kw_ref[...], 0, S)

    scale = 1.0 / (DH ** 0.5)
    qi = qb * bq + jax.lax.broadcasted_iota(jnp.int32, (bq, bk), 0)
    ki_loc = jax.lax.broadcasted_iota(jnp.int32, (bq, bk), 1)

    for h in range(gh):
        q = _norm_rope(q_ref[:, h * DH:(h + 1) * DH], qw_ref[...],
                       qb * bq, bq) * scale

        def body(kc, carry):
            m, l, acc = carry
            kc_rows = pl.ds(kc * bk, bk)
            s = jax.lax.dot_general(q, kn_ref[kc_rows, :],
                                    (((1,), (1,)), ((), ())),
                                    preferred_element_type=jnp.float32)
            s = jnp.where(kc * bk + ki_loc <= qi, s, -1e30)
            m_new = jnp.maximum(m, jnp.max(s, axis=-1, keepdims=True))
            alpha = jnp.exp(m - m_new)
            p = jnp.exp(s - m_new)
            l_new = l * alpha + jnp.sum(p, axis=-1, keepdims=True)
            acc_new = acc * alpha + jnp.dot(p, v_ref[kc_rows, :],
                                            preferred_element_type=jnp.float32)
            return m_new, l_new, acc_new

        init = (jnp.full((bq, 1), -1e30, jnp.float32),
                jnp.zeros((bq, 1), jnp.float32),
                jnp.zeros((bq, DH), jnp.float32))
        _, l, acc = jax.lax.fori_loop(0, qb + 1, body, init)
        o_ref[:, h * DH:(h + 1) * DH] = acc / l


def _flash(qkv, qnw, knw, bq=512, bk=512):
    nq = S // bq
    gh = H // KVH
    kern = functools.partial(_flash_kernel, bq=bq, bk=bk)
    return pl.pallas_call(
        kern,
        grid=(KVH, nq),
        in_specs=[
            pl.BlockSpec((bq, gh * DH), lambda g, qb: (qb, g)),
            pl.BlockSpec((S, DH), lambda g, qb: (0, H + g)),
            pl.BlockSpec((S, DH), lambda g, qb: (0, H + KVH + g)),
            pl.BlockSpec((1, DH), lambda g, qb: (0, 0)),
            pl.BlockSpec((1, DH), lambda g, qb: (0, 0)),
        ],
        out_specs=pl.BlockSpec((bq, gh * DH), lambda g, qb: (qb, g)),
        out_shape=jax.ShapeDtypeStruct((S, H * DH), jnp.float32),
        scratch_shapes=[
            pltpu.VMEM((S, DH), jnp.float32),
        ],
    )(qkv, qkv, qkv, qnw, knw)


def _oproj_kernel(a_ref, x_ref, ow_ref, pw_ref, rw_ref,
                  res_ref, hn_ref, lg_ref):
    ao = jnp.dot(a_ref[...], ow_ref[...], precision=None,
                 preferred_element_type=jnp.float32)
    h = x_ref[...] + ao
    res_ref[...] = h
    hn = _rms(h, pw_ref[...])
    hn_ref[...] = hn
    lg_ref[...] = jnp.dot(hn, rw_ref[...], precision=None,
                          preferred_element_type=jnp.float32)


def _oproj(attn, x, ow, pw, rw, br=256):
    nr = S // br
    return pl.pallas_call(
        _oproj_kernel,
        grid=(nr,),
        in_specs=[
            pl.BlockSpec((br, H * DH), lambda r: (r, 0)),
            pl.BlockSpec((br, D), lambda r: (r, 0)),
            pl.BlockSpec((H * DH, D), lambda r: (0, 0)),
            pl.BlockSpec((1, D), lambda r: (0, 0)),
            pl.BlockSpec((D, E), lambda r: (0, 0)),
        ],
        out_specs=[
            pl.BlockSpec((br, D), lambda r: (r, 0)),
            pl.BlockSpec((br, D), lambda r: (r, 0)),
            pl.BlockSpec((br, E), lambda r: (r, 0)),
        ],
        out_shape=[
            jax.ShapeDtypeStruct((S, D), jnp.float32),
            jax.ShapeDtypeStruct((S, D), jnp.float32),
            jax.ShapeDtypeStruct((S, E), jnp.float32),
        ],
    )(attn, x, ow, pw, rw)


def _moe_kernel(hn_ref, lg_ref, res_ref, g_ref, u_ref, d_ref,
                o_ref, acc_ref, *, bt):
    e = pl.program_id(1)

    x = hn_ref[...]
    g = jnp.dot(x, g_ref[0], preferred_element_type=jnp.float32)
    u = jnp.dot(x, u_ref[0], preferred_element_type=jnp.float32)
    hf = (g * jax.nn.sigmoid(g)) * u

    lg = lg_ref[...]
    mx = jnp.max(lg, axis=-1, keepdims=True)
    p = jnp.exp(lg - mx)
    p = p / jnp.sum(p, axis=-1, keepdims=True)
    v1 = jnp.max(p, axis=-1, keepdims=True)
    v2 = jnp.max(jnp.where(p >= v1, -1.0, p), axis=-1, keepdims=True)
    wsel = jnp.where(p >= v2, p, 0.0) / (v1 + v2)
    eid = jax.lax.broadcasted_iota(jnp.int32, (1, E), 1)
    w_e = jnp.sum(wsel * (eid == e).astype(jnp.float32), axis=-1,
                  keepdims=True)

    contrib = jnp.dot((hf * w_e).astype(jnp.bfloat16), d_ref[0],
                      preferred_element_type=jnp.float32)

    @pl.when(e == 0)
    def _():
        acc_ref[...] = contrib

    @pl.when(e > 0)
    def _():
        acc_ref[...] = acc_ref[...] + contrib

    @pl.when(e == E - 1)
    def _():
        o_ref[...] = acc_ref[...] + res_ref[...]


def _moe(hn_bf16, logits, res, gk, uk, dk, bt=512):
    nt = S // bt
    kern = functools.partial(_moe_kernel, bt=bt)
    return pl.pallas_call(
        kern,
        grid=(nt, E),
        in_specs=[
            pl.BlockSpec((bt, D), lambda t, e: (t, 0)),
            pl.BlockSpec((bt, E), lambda t, e: (t, 0)),
            pl.BlockSpec((bt, D), lambda t, e: (t, 0)),
            pl.BlockSpec((1, D, F), lambda t, e: (e, 0, 0)),
            pl.BlockSpec((1, D, F), lambda t, e: (e, 0, 0)),
            pl.BlockSpec((1, F, D), lambda t, e: (e, 0, 0)),
        ],
        out_specs=pl.BlockSpec((bt, D), lambda t, e: (t, 0)),
        out_shape=jax.ShapeDtypeStruct((S, D), jnp.float32),
        scratch_shapes=[pltpu.VMEM((bt, D), jnp.float32)],
    )(hn_bf16, logits, res, gk, uk, dk)


def kernel(hidden_states, input_ln_w, q_w, k_w, v_w, o_w, q_norm_w,
           k_norm_w, post_ln_w, router_w, gate_k, up_k, down_k):
    x = hidden_states.reshape(S, D)
    wqkv = jnp.concatenate([q_w, k_w, v_w], axis=1)

    qkv = _qkv(x, input_ln_w.reshape(1, D), wqkv)
    attn = _flash(qkv, q_norm_w.reshape(1, DH), k_norm_w.reshape(1, DH))
    res2, hn, logits = _oproj(attn, x, o_w, post_ln_w.reshape(1, D), router_w)
    out = _moe(hn.astype(jnp.bfloat16), logits, res2,
               gate_k.astype(jnp.bfloat16), up_k.astype(jnp.bfloat16),
               down_k.astype(jnp.bfloat16))
    return out.reshape(1, S, D)
